# Initial kernel scaffold; baseline (speedup 1.0000x reference)
import jax
import jax.numpy as jnp
from jax.experimental import pallas as pl


def kernel(feat1, feat2, W1, b1, W2, b2, Wp1, bp1, Wp2, bp2, edge_index1, edge_index2):
    raise NotImplementedError("write your pallas kernel here")


# speedup vs baseline: 3.3631x; 3.3631x over previous
import functools

import jax
import jax.numpy as jnp
from jax import lax
from jax.experimental import pallas as pl
from jax.experimental.pallas import tpu as pltpu
from jax.experimental.pallas import tpu_sc as plsc

N = 10000
E = 320000
D = 128
NS = 16
K = 80

EPT_AGG = E // NS
R_CHUNK = 624
R_TAIL = N - 15 * R_CHUNK
R_TC = 1000

_mesh = plsc.VectorSubcoreMesh(core_axis_name="c", subcore_axis_name="s")


def _rowsplit_copy(copy_fn, s):
    @pl.when(s < 15)
    def _():
        copy_fn(s * R_CHUNK, R_CHUNK)

    @pl.when(s == 15)
    def _():
        copy_fn(15 * R_CHUNK, R_TAIL)


@functools.partial(
    pl.kernel,
    mesh=_mesh,
    out_type=jax.ShapeDtypeStruct((2 * N, D), jnp.float32),
    scratch_types=[pltpu.VMEM((K,), jnp.int32),
                   pltpu.VMEM((K,), jnp.int32),
                   pltpu.VMEM((K, D), jnp.float32),
                   pltpu.VMEM((K, D), jnp.float32),
                   pltpu.VMEM_SHARED((N, D), jnp.float32)],
)
def _sc_degrees(src_hbm, dst_hbm, e0_hbm, e1_hbm, zeros_hbm, deg_hbm,
                sidx_v, didx_v, e0_v, e1_v, acc):
    c = lax.axis_index("c")
    s = lax.axis_index("s")

    def zero(r0, n):
        pltpu.sync_copy(zeros_hbm.at[pl.ds(r0, n)], acc.at[pl.ds(r0, n)])

    _rowsplit_copy(zero, s)
    pltpu.sync_copy(e0_hbm, e0_v)
    pltpu.sync_copy(e1_hbm, e1_v)
    plsc.subcore_barrier()

    base = c * E + s * EPT_AGG

    def body(j, carry):
        b = base + j * K
        pltpu.sync_copy(src_hbm.at[pl.ds(b, K)], sidx_v)
        pltpu.sync_copy(dst_hbm.at[pl.ds(b, K)], didx_v)
        pltpu.sync_copy(e0_v, acc.at[sidx_v], add=True)
        pltpu.sync_copy(e1_v, acc.at[didx_v], add=True)
        return carry

    lax.fori_loop(0, EPT_AGG // K, body, None)
    plsc.subcore_barrier()

    def writeout(r0, n):
        pltpu.sync_copy(acc.at[pl.ds(r0, n)],
                        deg_hbm.at[pl.ds(c * N + r0, n)])

    _rowsplit_copy(writeout, s)


@functools.partial(
    pl.kernel,
    mesh=_mesh,
    out_type=jax.ShapeDtypeStruct((2 * N, D), jnp.float32),
    scratch_types=[pltpu.VMEM((K,), jnp.int32),
                   pltpu.VMEM((K,), jnp.int32),
                   pltpu.VMEM((K, D), jnp.float32),
                   pltpu.VMEM_SHARED((N, D), jnp.float32),
                   pltpu.SemaphoreType.DMA],
)
def _sc_aggregate(x_hbm, gsrc_hbm, sdst_hbm, zeros_hbm, out_hbm,
                  sidx_v, didx_v, rows_v, acc, sem):
    c = lax.axis_index("c")
    s = lax.axis_index("s")

    def zero(r0, n):
        pltpu.sync_copy(zeros_hbm.at[pl.ds(r0, n)], acc.at[pl.ds(r0, n)])

    _rowsplit_copy(zero, s)
    plsc.subcore_barrier()

    base = c * E + s * EPT_AGG

    def body(j, carry):
        b = base + j * K
        pltpu.sync_copy(gsrc_hbm.at[pl.ds(b, K)], sidx_v)
        pltpu.sync_copy(sdst_hbm.at[pl.ds(b, K)], didx_v)
        pltpu.async_copy(x_hbm.at[sidx_v], rows_v, sem).wait()
        pltpu.sync_copy(rows_v, acc.at[didx_v], add=True)
        return carry

    lax.fori_loop(0, EPT_AGG // K, body, None)
    plsc.subcore_barrier()

    def writeout(r0, n):
        pltpu.sync_copy(acc.at[pl.ds(r0, n)],
                        out_hbm.at[pl.ds(c * N + r0, n)])

    _rowsplit_copy(writeout, s)


def _tc_prescale_body(x_ref, deg_ref, o_ref):
    dinv = lax.rsqrt(jnp.maximum(deg_ref[:, 0:1], 1.0))
    o_ref[:, :] = x_ref[:, :] * dinv


def _tc_prescale(x, degc):
    return pl.pallas_call(
        _tc_prescale_body,
        grid=((2 * N) // R_TC,),
        in_specs=[pl.BlockSpec((R_TC, D), lambda i: (i, 0)),
                  pl.BlockSpec((R_TC, D), lambda i: (i, 0))],
        out_specs=pl.BlockSpec((R_TC, D), lambda i: (i, 0)),
        out_shape=jax.ShapeDtypeStruct((2 * N, D), jnp.float32),
    )(x, degc)


def _tc_mid_body(a_ref, deg_ref, w1_ref, b1_ref, w2_ref, o_ref):
    din = lax.rsqrt(jnp.maximum(deg_ref[:, 1:2], 1.0))
    a = a_ref[:, :] * din
    h = jnp.dot(a, w1_ref[:, :], preferred_element_type=jnp.float32)
    h = jnp.maximum(h + b1_ref[:, :], 0.0)
    p = jnp.dot(h, w2_ref[:, :], preferred_element_type=jnp.float32)
    dout = lax.rsqrt(jnp.maximum(deg_ref[:, 0:1], 1.0))
    o_ref[:, :] = p * dout


def _tc_mid(agg1, degc, W1, b1, W2):
    return pl.pallas_call(
        _tc_mid_body,
        grid=((2 * N) // R_TC,),
        in_specs=[pl.BlockSpec((R_TC, D), lambda i: (i, 0)),
                  pl.BlockSpec((R_TC, D), lambda i: (i, 0)),
                  pl.BlockSpec((D, 2 * D), lambda i: (0, 0)),
                  pl.BlockSpec((1, 2 * D), lambda i: (0, 0)),
                  pl.BlockSpec((2 * D, D), lambda i: (0, 0))],
        out_specs=pl.BlockSpec((R_TC, D), lambda i: (i, 0)),
        out_shape=jax.ShapeDtypeStruct((2 * N, D), jnp.float32),
    )(agg1, degc, W1, b1, W2)


def _tc_final_body(a_ref, deg_ref, b2_ref, wp1_ref, bp1_ref, wp2_ref,
                   bp2_ref, o_ref):
    din = lax.rsqrt(jnp.maximum(deg_ref[:, 1:2], 1.0))
    h2 = jnp.maximum(a_ref[:, :] * din + b2_ref[:, :], 0.0)
    t = jnp.dot(h2, wp1_ref[:, :], preferred_element_type=jnp.float32)
    t = t + bp1_ref[:, :]
    t = jnp.where(t > 0.0, t, jnp.exp(t) - 1.0)
    z = jnp.dot(t, wp2_ref[:, :], preferred_element_type=jnp.float32)
    o_ref[:, :] = z + bp2_ref[:, :]


def _tc_final(agg2, degc, b2, Wp1, bp1, Wp2, bp2):
    return pl.pallas_call(
        _tc_final_body,
        grid=((2 * N) // R_TC,),
        in_specs=[pl.BlockSpec((R_TC, D), lambda i: (i, 0)),
                  pl.BlockSpec((R_TC, D), lambda i: (i, 0)),
                  pl.BlockSpec((1, D), lambda i: (0, 0)),
                  pl.BlockSpec((D, D), lambda i: (0, 0)),
                  pl.BlockSpec((1, D), lambda i: (0, 0)),
                  pl.BlockSpec((D, D), lambda i: (0, 0)),
                  pl.BlockSpec((1, D), lambda i: (0, 0))],
        out_specs=pl.BlockSpec((R_TC, D), lambda i: (i, 0)),
        out_shape=jax.ShapeDtypeStruct((2 * N, D), jnp.float32),
    )(agg2, degc, b2, Wp1, bp1, Wp2, bp2)


def kernel(feat1, feat2, W1, b1, W2, b2, Wp1, bp1, Wp2, bp2,
           edge_index1, edge_index2):
    src1, dst1 = edge_index1[0], edge_index1[1]
    src2, dst2 = edge_index2[0], edge_index2[1]
    gsrc = jnp.concatenate([src1, src2 + N])
    ssrc = jnp.concatenate([src1, src2])
    sdst = jnp.concatenate([dst1, dst2])
    x_st = jnp.concatenate([feat1, feat2], axis=0)
    zerosD = jnp.zeros((N, D), jnp.float32)
    lane = jnp.arange(D, dtype=jnp.int32)[None, :]
    e0 = jnp.broadcast_to((lane == 0).astype(jnp.float32), (K, D))
    e1 = jnp.broadcast_to((lane == 1).astype(jnp.float32), (K, D))

    degc = _sc_degrees(ssrc, sdst, e0, e1, zerosD)
    x_scaled = _tc_prescale(x_st, degc)
    agg1 = _sc_aggregate(x_scaled, gsrc, sdst, zerosD)
    pre2 = _tc_mid(agg1, degc, W1, b1.reshape(1, -1), W2)
    agg2 = _sc_aggregate(pre2, gsrc, sdst, zerosD)
    z = _tc_final(agg2, degc, b2.reshape(1, -1), Wp1, bp1.reshape(1, -1),
                  Wp2, bp2.reshape(1, -1))
    return (z[:N], z[N:])


# speedup vs baseline: 6.9903x; 2.0785x over previous
import functools

import jax
import jax.numpy as jnp
from jax import lax
from jax.experimental import pallas as pl
from jax.experimental.pallas import tpu as pltpu
from jax.experimental.pallas import tpu_sc as plsc

N = 10000
E = 320000
D = 128
NS = 16
K = 80

EPT_AGG = E // NS
NCH = EPT_AGG // K
GSC = 50
NSC = NCH // GSC
R_CHUNK = 624
R_TAIL = N - 15 * R_CHUNK
R_TC = 1000

_mesh = plsc.VectorSubcoreMesh(core_axis_name="c", subcore_axis_name="s")


def _rowsplit_copy(copy_fn, s):
    @pl.when(s < 15)
    def _():
        copy_fn(s * R_CHUNK, R_CHUNK)

    @pl.when(s == 15)
    def _():
        copy_fn(15 * R_CHUNK, R_TAIL)


@functools.partial(
    pl.kernel,
    mesh=_mesh,
    out_type=jax.ShapeDtypeStruct((2 * N, D), jnp.float32),
    scratch_types=[pltpu.VMEM((GSC, K), jnp.int32),
                   pltpu.VMEM((GSC, K), jnp.int32),
                   pltpu.VMEM((K, D), jnp.float32),
                   pltpu.VMEM((K, D), jnp.float32),
                   pltpu.VMEM_SHARED((N, D), jnp.float32)],
)
def _sc_degrees(src_hbm, dst_hbm, e0_hbm, e1_hbm, zeros_hbm, deg_hbm,
                sidx_v, didx_v, e0_v, e1_v, acc):
    c = lax.axis_index("c")
    s = lax.axis_index("s")

    def zero(r0, n):
        pltpu.sync_copy(zeros_hbm.at[pl.ds(r0, n)], acc.at[pl.ds(r0, n)])

    _rowsplit_copy(zero, s)
    pltpu.sync_copy(e0_hbm, e0_v)
    pltpu.sync_copy(e1_hbm, e1_v)
    plsc.subcore_barrier()

    def outer(g, carry):
        pltpu.sync_copy(src_hbm.at[c, s, g], sidx_v)
        pltpu.sync_copy(dst_hbm.at[c, s, g], didx_v)

        def body(j, carry2):
            pltpu.sync_copy(e0_v, acc.at[sidx_v.at[j]], add=True)
            pltpu.sync_copy(e1_v, acc.at[didx_v.at[j]], add=True)
            return carry2

        lax.fori_loop(0, GSC, body, None)
        return carry

    lax.fori_loop(0, NSC, outer, None)
    plsc.subcore_barrier()

    def writeout(r0, n):
        pltpu.sync_copy(acc.at[pl.ds(r0, n)],
                        deg_hbm.at[pl.ds(c * N + r0, n)])

    _rowsplit_copy(writeout, s)


@functools.partial(
    pl.kernel,
    mesh=_mesh,
    out_type=jax.ShapeDtypeStruct((2 * N, D), jnp.float32),
    scratch_types=[pltpu.VMEM((GSC, K), jnp.int32),
                   pltpu.VMEM((GSC, K), jnp.int32),
                   pltpu.VMEM((K, D), jnp.float32),
                   pltpu.VMEM((K, D), jnp.float32),
                   pltpu.VMEM_SHARED((N, D), jnp.float32),
                   pltpu.SemaphoreType.DMA,
                   pltpu.SemaphoreType.DMA],
)
def _sc_aggregate(x_hbm, gsrc_hbm, sdst_hbm, zeros_hbm, out_hbm,
                  sidx_v, didx_v, rows0_v, rows1_v, acc, sem0, sem1):
    c = lax.axis_index("c")
    s = lax.axis_index("s")

    def zero(r0, n):
        pltpu.sync_copy(zeros_hbm.at[pl.ds(r0, n)], acc.at[pl.ds(r0, n)])

    _rowsplit_copy(zero, s)
    plsc.subcore_barrier()

    def gather(j, rows_v, sem):
        pltpu.async_copy(x_hbm.at[sidx_v.at[j]], rows_v, sem)

    def gather_wait(j, rows_v, sem):
        pltpu.make_async_copy(x_hbm.at[sidx_v.at[j]], rows_v, sem).wait()

    def outer(g, carry):
        pltpu.sync_copy(gsrc_hbm.at[c, s, g], sidx_v)
        pltpu.sync_copy(sdst_hbm.at[c, s, g], didx_v)
        gather(0, rows0_v, sem0)

        def body(j, carry2):
            even = j % 2 == 0

            @pl.when((j + 1 < GSC) & even)
            def _():
                gather(j + 1, rows1_v, sem1)

            @pl.when((j + 1 < GSC) & (~even))
            def _():
                gather(j + 1, rows0_v, sem0)

            @pl.when(even)
            def _():
                gather_wait(j, rows0_v, sem0)
                pltpu.sync_copy(rows0_v, acc.at[didx_v.at[j]], add=True)

            @pl.when(~even)
            def _():
                gather_wait(j, rows1_v, sem1)
                pltpu.sync_copy(rows1_v, acc.at[didx_v.at[j]], add=True)

            return carry2

        lax.fori_loop(0, GSC, body, None)
        return carry

    lax.fori_loop(0, NSC, outer, None)
    plsc.subcore_barrier()

    def writeout(r0, n):
        pltpu.sync_copy(acc.at[pl.ds(r0, n)],
                        out_hbm.at[pl.ds(c * N + r0, n)])

    _rowsplit_copy(writeout, s)


def _tc_prescale_body(x_ref, deg_ref, o_ref):
    dinv = lax.rsqrt(jnp.maximum(deg_ref[:, 0:1], 1.0))
    o_ref[:, :] = x_ref[:, :] * dinv


def _tc_prescale(x, degc):
    return pl.pallas_call(
        _tc_prescale_body,
        grid=((2 * N) // R_TC,),
        in_specs=[pl.BlockSpec((R_TC, D), lambda i: (i, 0)),
                  pl.BlockSpec((R_TC, D), lambda i: (i, 0))],
        out_specs=pl.BlockSpec((R_TC, D), lambda i: (i, 0)),
        out_shape=jax.ShapeDtypeStruct((2 * N, D), jnp.float32),
    )(x, degc)


def _tc_mid_body(a_ref, deg_ref, w1_ref, b1_ref, w2_ref, o_ref):
    din = lax.rsqrt(jnp.maximum(deg_ref[:, 1:2], 1.0))
    a = a_ref[:, :] * din
    h = jnp.dot(a, w1_ref[:, :], preferred_element_type=jnp.float32)
    h = jnp.maximum(h + b1_ref[:, :], 0.0)
    p = jnp.dot(h, w2_ref[:, :], preferred_element_type=jnp.float32)
    dout = lax.rsqrt(jnp.maximum(deg_ref[:, 0:1], 1.0))
    o_ref[:, :] = p * dout


def _tc_mid(agg1, degc, W1, b1, W2):
    return pl.pallas_call(
        _tc_mid_body,
        grid=((2 * N) // R_TC,),
        in_specs=[pl.BlockSpec((R_TC, D), lambda i: (i, 0)),
                  pl.BlockSpec((R_TC, D), lambda i: (i, 0)),
                  pl.BlockSpec((D, 2 * D), lambda i: (0, 0)),
                  pl.BlockSpec((1, 2 * D), lambda i: (0, 0)),
                  pl.BlockSpec((2 * D, D), lambda i: (0, 0))],
        out_specs=pl.BlockSpec((R_TC, D), lambda i: (i, 0)),
        out_shape=jax.ShapeDtypeStruct((2 * N, D), jnp.float32),
    )(agg1, degc, W1, b1, W2)


def _tc_final_body(a_ref, deg_ref, b2_ref, wp1_ref, bp1_ref, wp2_ref,
                   bp2_ref, o_ref):
    din = lax.rsqrt(jnp.maximum(deg_ref[:, 1:2], 1.0))
    h2 = jnp.maximum(a_ref[:, :] * din + b2_ref[:, :], 0.0)
    t = jnp.dot(h2, wp1_ref[:, :], preferred_element_type=jnp.float32)
    t = t + bp1_ref[:, :]
    t = jnp.where(t > 0.0, t, jnp.exp(t) - 1.0)
    z = jnp.dot(t, wp2_ref[:, :], preferred_element_type=jnp.float32)
    o_ref[:, :] = z + bp2_ref[:, :]


def _tc_final(agg2, degc, b2, Wp1, bp1, Wp2, bp2):
    return pl.pallas_call(
        _tc_final_body,
        grid=((2 * N) // R_TC,),
        in_specs=[pl.BlockSpec((R_TC, D), lambda i: (i, 0)),
                  pl.BlockSpec((R_TC, D), lambda i: (i, 0)),
                  pl.BlockSpec((1, D), lambda i: (0, 0)),
                  pl.BlockSpec((D, D), lambda i: (0, 0)),
                  pl.BlockSpec((1, D), lambda i: (0, 0)),
                  pl.BlockSpec((D, D), lambda i: (0, 0)),
                  pl.BlockSpec((1, D), lambda i: (0, 0))],
        out_specs=pl.BlockSpec((R_TC, D), lambda i: (i, 0)),
        out_shape=jax.ShapeDtypeStruct((2 * N, D), jnp.float32),
    )(agg2, degc, b2, Wp1, bp1, Wp2, bp2)


def kernel(feat1, feat2, W1, b1, W2, b2, Wp1, bp1, Wp2, bp2,
           edge_index1, edge_index2):
    src1, dst1 = edge_index1[0], edge_index1[1]
    src2, dst2 = edge_index2[0], edge_index2[1]
    idx4 = (2, NS, NSC, GSC, K)
    gsrc = jnp.concatenate([src1, src2 + N]).reshape(idx4)
    ssrc = jnp.concatenate([src1, src2]).reshape(idx4)
    sdst = jnp.concatenate([dst1, dst2]).reshape(idx4)
    x_st = jnp.concatenate([feat1, feat2], axis=0)
    zerosD = jnp.zeros((N, D), jnp.float32)
    lane = jnp.arange(D, dtype=jnp.int32)[None, :]
    e0 = jnp.broadcast_to((lane == 0).astype(jnp.float32), (K, D))
    e1 = jnp.broadcast_to((lane == 1).astype(jnp.float32), (K, D))

    degc = _sc_degrees(ssrc, sdst, e0, e1, zerosD)
    x_scaled = _tc_prescale(x_st, degc)
    agg1 = _sc_aggregate(x_scaled, gsrc, sdst, zerosD)
    pre2 = _tc_mid(agg1, degc, W1, b1.reshape(1, -1), W2)
    agg2 = _sc_aggregate(pre2, gsrc, sdst, zerosD)
    z = _tc_final(agg2, degc, b2.reshape(1, -1), Wp1, bp1.reshape(1, -1),
                  Wp2, bp2.reshape(1, -1))
    return (z[:N], z[N:])


# speedup vs baseline: 7.4724x; 1.0690x over previous
import functools

import jax
import jax.numpy as jnp
from jax import lax
from jax.experimental import pallas as pl
from jax.experimental.pallas import tpu as pltpu
from jax.experimental.pallas import tpu_sc as plsc

N = 10000
E = 320000
D = 128
NS = 16
K = 80

EPT_AGG = E // NS
NCH = EPT_AGG // K
GSC = 25
NSC = NCH // GSC
R_CHUNK = 624
R_TAIL = N - 15 * R_CHUNK
R_TC = 1000

_mesh = plsc.VectorSubcoreMesh(core_axis_name="c", subcore_axis_name="s")


def _rowsplit_copy(copy_fn, s):
    @pl.when(s < 15)
    def _():
        copy_fn(s * R_CHUNK, R_CHUNK)

    @pl.when(s == 15)
    def _():
        copy_fn(15 * R_CHUNK, R_TAIL)


@functools.partial(
    pl.kernel,
    mesh=_mesh,
    out_type=jax.ShapeDtypeStruct((2 * N, D), jnp.float32),
    scratch_types=[pltpu.VMEM((GSC, K), jnp.int32),
                   pltpu.VMEM((GSC, K), jnp.int32),
                   pltpu.VMEM((K, D), jnp.float32),
                   pltpu.VMEM((K, D), jnp.float32),
                   pltpu.VMEM_SHARED((N, D), jnp.float32)],
)
def _sc_degrees(src_hbm, dst_hbm, e0_hbm, e1_hbm, zeros_hbm, deg_hbm,
                sidx_v, didx_v, e0_v, e1_v, acc):
    c = lax.axis_index("c")
    s = lax.axis_index("s")

    def zero(r0, n):
        pltpu.sync_copy(zeros_hbm.at[pl.ds(r0, n)], acc.at[pl.ds(r0, n)])

    _rowsplit_copy(zero, s)
    pltpu.sync_copy(e0_hbm, e0_v)
    pltpu.sync_copy(e1_hbm, e1_v)
    plsc.subcore_barrier()

    def outer(g, carry):
        pltpu.sync_copy(src_hbm.at[c, s, g], sidx_v)
        pltpu.sync_copy(dst_hbm.at[c, s, g], didx_v)

        def body(j, carry2):
            pltpu.sync_copy(e0_v, acc.at[sidx_v.at[j]], add=True)
            pltpu.sync_copy(e1_v, acc.at[didx_v.at[j]], add=True)
            return carry2

        lax.fori_loop(0, GSC, body, None)
        return carry

    lax.fori_loop(0, NSC, outer, None)
    plsc.subcore_barrier()

    def writeout(r0, n):
        pltpu.sync_copy(acc.at[pl.ds(r0, n)],
                        deg_hbm.at[pl.ds(c * N + r0, n)])

    _rowsplit_copy(writeout, s)


@functools.partial(
    pl.kernel,
    mesh=_mesh,
    out_type=jax.ShapeDtypeStruct((2 * N, D), jnp.float32),
    scratch_types=[pltpu.VMEM((GSC, K), jnp.int32),
                   pltpu.VMEM((GSC, K), jnp.int32),
                   pltpu.VMEM((K, D), jnp.float32),
                   pltpu.VMEM((K, D), jnp.float32),
                   pltpu.VMEM((K, D), jnp.float32),
                   pltpu.VMEM((K, D), jnp.float32),
                   pltpu.VMEM_SHARED((N, D), jnp.float32),
                   pltpu.SemaphoreType.DMA,
                   pltpu.SemaphoreType.DMA,
                   pltpu.SemaphoreType.DMA,
                   pltpu.SemaphoreType.DMA],
)
def _sc_aggregate(x_hbm, gsrc_hbm, sdst_hbm, zeros_hbm, out_hbm,
                  sidx_v, didx_v, rows0_v, rows1_v, rows2_v, rows3_v, acc,
                  sem0, sem1, sem2, sem3):
    c = lax.axis_index("c")
    s = lax.axis_index("s")
    rows = (rows0_v, rows1_v, rows2_v, rows3_v)
    sems = (sem0, sem1, sem2, sem3)

    def zero(r0, n):
        pltpu.sync_copy(zeros_hbm.at[pl.ds(r0, n)], acc.at[pl.ds(r0, n)])

    _rowsplit_copy(zero, s)
    plsc.subcore_barrier()

    def gather(j, p):
        pltpu.async_copy(x_hbm.at[sidx_v.at[j]], rows[p], sems[p])

    def gather_wait(j, p):
        pltpu.make_async_copy(x_hbm.at[sidx_v.at[j]], rows[p], sems[p]).wait()

    def outer(g, carry):
        pltpu.sync_copy(gsrc_hbm.at[c, s, g], sidx_v)
        pltpu.sync_copy(sdst_hbm.at[c, s, g], didx_v)
        for p in range(3):
            gather(p, p)

        def body(j, carry2):
            for q in range(4):
                @pl.when((j + 3 < GSC) & ((j + 3) % 4 == q))
                def _(q=q):
                    gather(j + 3, q)

            for q in range(4):
                @pl.when(j % 4 == q)
                def _(q=q):
                    gather_wait(j, q)
                    pltpu.sync_copy(rows[q], acc.at[didx_v.at[j]], add=True)
            return carry2

        lax.fori_loop(0, GSC, body, None)
        return carry

    lax.fori_loop(0, NSC, outer, None)
    plsc.subcore_barrier()

    def writeout(r0, n):
        pltpu.sync_copy(acc.at[pl.ds(r0, n)],
                        out_hbm.at[pl.ds(c * N + r0, n)])

    _rowsplit_copy(writeout, s)


def _tc_prescale_body(x_ref, deg_ref, o_ref):
    dinv = lax.rsqrt(jnp.maximum(deg_ref[:, 0:1], 1.0))
    o_ref[:, :] = x_ref[:, :] * dinv


def _tc_prescale(x, degc):
    return pl.pallas_call(
        _tc_prescale_body,
        grid=((2 * N) // R_TC,),
        in_specs=[pl.BlockSpec((R_TC, D), lambda i: (i, 0)),
                  pl.BlockSpec((R_TC, D), lambda i: (i, 0))],
        out_specs=pl.BlockSpec((R_TC, D), lambda i: (i, 0)),
        out_shape=jax.ShapeDtypeStruct((2 * N, D), jnp.float32),
    )(x, degc)


def _tc_mid_body(a_ref, deg_ref, w1_ref, b1_ref, w2_ref, o_ref):
    din = lax.rsqrt(jnp.maximum(deg_ref[:, 1:2], 1.0))
    a = a_ref[:, :] * din
    h = jnp.dot(a, w1_ref[:, :], preferred_element_type=jnp.float32)
    h = jnp.maximum(h + b1_ref[:, :], 0.0)
    p = jnp.dot(h, w2_ref[:, :], preferred_element_type=jnp.float32)
    dout = lax.rsqrt(jnp.maximum(deg_ref[:, 0:1], 1.0))
    o_ref[:, :] = p * dout


def _tc_mid(agg1, degc, W1, b1, W2):
    return pl.pallas_call(
        _tc_mid_body,
        grid=((2 * N) // R_TC,),
        in_specs=[pl.BlockSpec((R_TC, D), lambda i: (i, 0)),
                  pl.BlockSpec((R_TC, D), lambda i: (i, 0)),
                  pl.BlockSpec((D, 2 * D), lambda i: (0, 0)),
                  pl.BlockSpec((1, 2 * D), lambda i: (0, 0)),
                  pl.BlockSpec((2 * D, D), lambda i: (0, 0))],
        out_specs=pl.BlockSpec((R_TC, D), lambda i: (i, 0)),
        out_shape=jax.ShapeDtypeStruct((2 * N, D), jnp.float32),
    )(agg1, degc, W1, b1, W2)


def _tc_final_body(a_ref, deg_ref, b2_ref, wp1_ref, bp1_ref, wp2_ref,
                   bp2_ref, o_ref):
    din = lax.rsqrt(jnp.maximum(deg_ref[:, 1:2], 1.0))
    h2 = jnp.maximum(a_ref[:, :] * din + b2_ref[:, :], 0.0)
    t = jnp.dot(h2, wp1_ref[:, :], preferred_element_type=jnp.float32)
    t = t + bp1_ref[:, :]
    t = jnp.where(t > 0.0, t, jnp.exp(t) - 1.0)
    z = jnp.dot(t, wp2_ref[:, :], preferred_element_type=jnp.float32)
    o_ref[:, :] = z + bp2_ref[:, :]


def _tc_final(agg2, degc, b2, Wp1, bp1, Wp2, bp2):
    return pl.pallas_call(
        _tc_final_body,
        grid=((2 * N) // R_TC,),
        in_specs=[pl.BlockSpec((R_TC, D), lambda i: (i, 0)),
                  pl.BlockSpec((R_TC, D), lambda i: (i, 0)),
                  pl.BlockSpec((1, D), lambda i: (0, 0)),
                  pl.BlockSpec((D, D), lambda i: (0, 0)),
                  pl.BlockSpec((1, D), lambda i: (0, 0)),
                  pl.BlockSpec((D, D), lambda i: (0, 0)),
                  pl.BlockSpec((1, D), lambda i: (0, 0))],
        out_specs=pl.BlockSpec((R_TC, D), lambda i: (i, 0)),
        out_shape=jax.ShapeDtypeStruct((2 * N, D), jnp.float32),
    )(agg2, degc, b2, Wp1, bp1, Wp2, bp2)


def kernel(feat1, feat2, W1, b1, W2, b2, Wp1, bp1, Wp2, bp2,
           edge_index1, edge_index2):
    src1, dst1 = edge_index1[0], edge_index1[1]
    src2, dst2 = edge_index2[0], edge_index2[1]
    idx4 = (2, NS, NSC, GSC, K)
    gsrc = jnp.concatenate([src1, src2 + N]).reshape(idx4)
    ssrc = jnp.concatenate([src1, src2]).reshape(idx4)
    sdst = jnp.concatenate([dst1, dst2]).reshape(idx4)
    x_st = jnp.concatenate([feat1, feat2], axis=0)
    zerosD = jnp.zeros((N, D), jnp.float32)
    lane = jnp.arange(D, dtype=jnp.int32)[None, :]
    e0 = jnp.broadcast_to((lane == 0).astype(jnp.float32), (K, D))
    e1 = jnp.broadcast_to((lane == 1).astype(jnp.float32), (K, D))

    degc = _sc_degrees(ssrc, sdst, e0, e1, zerosD)
    x_scaled = _tc_prescale(x_st, degc)
    agg1 = _sc_aggregate(x_scaled, gsrc, sdst, zerosD)
    pre2 = _tc_mid(agg1, degc, W1, b1.reshape(1, -1), W2)
    agg2 = _sc_aggregate(pre2, gsrc, sdst, zerosD)
    z = _tc_final(agg2, degc, b2.reshape(1, -1), Wp1, bp1.reshape(1, -1),
                  Wp2, bp2.reshape(1, -1))
    return (z[:N], z[N:])


# speedup vs baseline: 7.5444x; 1.0096x over previous
import functools

import jax
import jax.numpy as jnp
from jax import lax
from jax.experimental import pallas as pl
from jax.experimental.pallas import tpu as pltpu
from jax.experimental.pallas import tpu_sc as plsc

N = 10000
E = 320000
D = 128
NS = 16
K = 80

EPT_AGG = E // NS
NCH = EPT_AGG // K
GSC = 25
NSC = NCH // GSC
R_CHUNK = 624
R_TAIL = N - 15 * R_CHUNK
R_TC = 1000

_mesh = plsc.VectorSubcoreMesh(core_axis_name="c", subcore_axis_name="s")


def _rowsplit_copy(copy_fn, s):
    @pl.when(s < 15)
    def _():
        copy_fn(s * R_CHUNK, R_CHUNK)

    @pl.when(s == 15)
    def _():
        copy_fn(15 * R_CHUNK, R_TAIL)


@functools.partial(
    pl.kernel,
    mesh=_mesh,
    out_type=jax.ShapeDtypeStruct((2 * N, D), jnp.float32),
    scratch_types=[pltpu.VMEM((GSC, K), jnp.int32),
                   pltpu.VMEM((GSC, K), jnp.int32),
                   pltpu.VMEM((K, D), jnp.float32),
                   pltpu.VMEM((K, D), jnp.float32),
                   pltpu.VMEM_SHARED((N, D), jnp.float32),
                   pltpu.SemaphoreType.DMA,
                   pltpu.SemaphoreType.DMA],
)
def _sc_degrees(src_hbm, dst_hbm, e0_hbm, e1_hbm, zeros_hbm, deg_hbm,
                sidx_v, didx_v, e0_v, e1_v, acc, sem0, sem1):
    c = lax.axis_index("c")
    s = lax.axis_index("s")
    sems = (sem0, sem1)

    def zero(r0, n):
        pltpu.sync_copy(zeros_hbm.at[pl.ds(r0, n)], acc.at[pl.ds(r0, n)])

    _rowsplit_copy(zero, s)
    pltpu.sync_copy(e0_hbm, e0_v)
    pltpu.sync_copy(e1_hbm, e1_v)
    plsc.subcore_barrier()

    def fire(j, p):
        pltpu.async_copy(e0_v, acc.at[sidx_v.at[j]], sems[p], add=True)
        pltpu.async_copy(e1_v, acc.at[didx_v.at[j]], sems[p], add=True)

    def drain(j, p):
        pltpu.make_async_copy(e0_v, acc.at[sidx_v.at[j]], sems[p]).wait()
        pltpu.make_async_copy(e1_v, acc.at[didx_v.at[j]], sems[p]).wait()

    def outer(g, carry):
        pltpu.sync_copy(src_hbm.at[c, s, g], sidx_v)
        pltpu.sync_copy(dst_hbm.at[c, s, g], didx_v)
        fire(0, 0)

        def body(j, carry2):
            for q in range(2):
                @pl.when((j + 1 < GSC) & ((j + 1) % 2 == q))
                def _(q=q):
                    fire(j + 1, q)

            for q in range(2):
                @pl.when(j % 2 == q)
                def _(q=q):
                    drain(j, q)
            return carry2

        lax.fori_loop(0, GSC, body, None)
        return carry

    lax.fori_loop(0, NSC, outer, None)
    plsc.subcore_barrier()

    def writeout(r0, n):
        pltpu.sync_copy(acc.at[pl.ds(r0, n)],
                        deg_hbm.at[pl.ds(c * N + r0, n)])

    _rowsplit_copy(writeout, s)


@functools.partial(
    pl.kernel,
    mesh=_mesh,
    out_type=jax.ShapeDtypeStruct((2 * N, D), jnp.float32),
    scratch_types=[pltpu.VMEM((GSC, K), jnp.int32),
                   pltpu.VMEM((GSC, K), jnp.int32),
                   pltpu.VMEM((K, D), jnp.float32),
                   pltpu.VMEM((K, D), jnp.float32),
                   pltpu.VMEM((K, D), jnp.float32),
                   pltpu.VMEM((K, D), jnp.float32),
                   pltpu.VMEM_SHARED((N, D), jnp.float32),
                   pltpu.SemaphoreType.DMA,
                   pltpu.SemaphoreType.DMA,
                   pltpu.SemaphoreType.DMA,
                   pltpu.SemaphoreType.DMA,
                   pltpu.SemaphoreType.DMA,
                   pltpu.SemaphoreType.DMA],
)
def _sc_aggregate(x_hbm, gsrc_hbm, sdst_hbm, zeros_hbm, out_hbm,
                  sidx_v, didx_v, rows0_v, rows1_v, rows2_v, rows3_v, acc,
                  sem0, sem1, sem2, sem3, ssem0, ssem1):
    c = lax.axis_index("c")
    s = lax.axis_index("s")
    rows = (rows0_v, rows1_v, rows2_v, rows3_v)
    sems = (sem0, sem1, sem2, sem3)
    ssems = (ssem0, ssem1)

    def zero(r0, n):
        pltpu.sync_copy(zeros_hbm.at[pl.ds(r0, n)], acc.at[pl.ds(r0, n)])

    _rowsplit_copy(zero, s)
    plsc.subcore_barrier()

    def gather(j, p):
        pltpu.async_copy(x_hbm.at[sidx_v.at[j]], rows[p], sems[p])

    def gather_wait(j, p):
        pltpu.make_async_copy(x_hbm.at[sidx_v.at[j]], rows[p], sems[p]).wait()

    def outer(g, carry):
        pltpu.sync_copy(gsrc_hbm.at[c, s, g], sidx_v)
        pltpu.sync_copy(sdst_hbm.at[c, s, g], didx_v)
        for p in range(3):
            gather(p, p)

        def body(j, carry2):
            for q in range(4):
                @pl.when((j >= 1) & ((j - 1) % 4 == q))
                def _(q=q):
                    pltpu.make_async_copy(rows[q], acc.at[didx_v.at[j - 1]],
                                          ssems[q % 2]).wait()

            for q in range(4):
                @pl.when((j + 3 < GSC) & ((j + 3) % 4 == q))
                def _(q=q):
                    gather(j + 3, q)

            for q in range(4):
                @pl.when(j % 4 == q)
                def _(q=q):
                    gather_wait(j, q)
                    pltpu.async_copy(rows[q], acc.at[didx_v.at[j]],
                                     ssems[q % 2], add=True)
            return carry2

        lax.fori_loop(0, GSC, body, None)
        for q in range(4):
            @pl.when((GSC - 1) % 4 == q)
            def _(q=q):
                pltpu.make_async_copy(rows[q], acc.at[didx_v.at[GSC - 1]],
                                      ssems[q % 2]).wait()
        return carry

    lax.fori_loop(0, NSC, outer, None)
    plsc.subcore_barrier()

    def writeout(r0, n):
        pltpu.sync_copy(acc.at[pl.ds(r0, n)],
                        out_hbm.at[pl.ds(c * N + r0, n)])

    _rowsplit_copy(writeout, s)


def _tc_prescale_body(x_ref, deg_ref, o_ref):
    dinv = lax.rsqrt(jnp.maximum(deg_ref[:, 0:1], 1.0))
    o_ref[:, :] = x_ref[:, :] * dinv


def _tc_prescale(x, degc):
    return pl.pallas_call(
        _tc_prescale_body,
        grid=((2 * N) // R_TC,),
        in_specs=[pl.BlockSpec((R_TC, D), lambda i: (i, 0)),
                  pl.BlockSpec((R_TC, D), lambda i: (i, 0))],
        out_specs=pl.BlockSpec((R_TC, D), lambda i: (i, 0)),
        out_shape=jax.ShapeDtypeStruct((2 * N, D), jnp.float32),
    )(x, degc)


def _tc_mid_body(a_ref, deg_ref, w1_ref, b1_ref, w2_ref, o_ref):
    din = lax.rsqrt(jnp.maximum(deg_ref[:, 1:2], 1.0))
    a = a_ref[:, :] * din
    h = jnp.dot(a, w1_ref[:, :], preferred_element_type=jnp.float32)
    h = jnp.maximum(h + b1_ref[:, :], 0.0)
    p = jnp.dot(h, w2_ref[:, :], preferred_element_type=jnp.float32)
    dout = lax.rsqrt(jnp.maximum(deg_ref[:, 0:1], 1.0))
    o_ref[:, :] = p * dout


def _tc_mid(agg1, degc, W1, b1, W2):
    return pl.pallas_call(
        _tc_mid_body,
        grid=((2 * N) // R_TC,),
        in_specs=[pl.BlockSpec((R_TC, D), lambda i: (i, 0)),
                  pl.BlockSpec((R_TC, D), lambda i: (i, 0)),
                  pl.BlockSpec((D, 2 * D), lambda i: (0, 0)),
                  pl.BlockSpec((1, 2 * D), lambda i: (0, 0)),
                  pl.BlockSpec((2 * D, D), lambda i: (0, 0))],
        out_specs=pl.BlockSpec((R_TC, D), lambda i: (i, 0)),
        out_shape=jax.ShapeDtypeStruct((2 * N, D), jnp.float32),
    )(agg1, degc, W1, b1, W2)


def _tc_final_body(a_ref, deg_ref, b2_ref, wp1_ref, bp1_ref, wp2_ref,
                   bp2_ref, o_ref):
    din = lax.rsqrt(jnp.maximum(deg_ref[:, 1:2], 1.0))
    h2 = jnp.maximum(a_ref[:, :] * din + b2_ref[:, :], 0.0)
    t = jnp.dot(h2, wp1_ref[:, :], preferred_element_type=jnp.float32)
    t = t + bp1_ref[:, :]
    t = jnp.where(t > 0.0, t, jnp.exp(t) - 1.0)
    z = jnp.dot(t, wp2_ref[:, :], preferred_element_type=jnp.float32)
    o_ref[:, :] = z + bp2_ref[:, :]


def _tc_final(agg2, degc, b2, Wp1, bp1, Wp2, bp2):
    return pl.pallas_call(
        _tc_final_body,
        grid=((2 * N) // R_TC,),
        in_specs=[pl.BlockSpec((R_TC, D), lambda i: (i, 0)),
                  pl.BlockSpec((R_TC, D), lambda i: (i, 0)),
                  pl.BlockSpec((1, D), lambda i: (0, 0)),
                  pl.BlockSpec((D, D), lambda i: (0, 0)),
                  pl.BlockSpec((1, D), lambda i: (0, 0)),
                  pl.BlockSpec((D, D), lambda i: (0, 0)),
                  pl.BlockSpec((1, D), lambda i: (0, 0))],
        out_specs=pl.BlockSpec((R_TC, D), lambda i: (i, 0)),
        out_shape=jax.ShapeDtypeStruct((2 * N, D), jnp.float32),
    )(agg2, degc, b2, Wp1, bp1, Wp2, bp2)


def kernel(feat1, feat2, W1, b1, W2, b2, Wp1, bp1, Wp2, bp2,
           edge_index1, edge_index2):
    src1, dst1 = edge_index1[0], edge_index1[1]
    src2, dst2 = edge_index2[0], edge_index2[1]
    idx4 = (2, NS, NSC, GSC, K)
    gsrc = jnp.concatenate([src1, src2 + N]).reshape(idx4)
    ssrc = jnp.concatenate([src1, src2]).reshape(idx4)
    sdst = jnp.concatenate([dst1, dst2]).reshape(idx4)
    x_st = jnp.concatenate([feat1, feat2], axis=0)
    zerosD = jnp.zeros((N, D), jnp.float32)
    lane = jnp.arange(D, dtype=jnp.int32)[None, :]
    e0 = jnp.broadcast_to((lane == 0).astype(jnp.float32), (K, D))
    e1 = jnp.broadcast_to((lane == 1).astype(jnp.float32), (K, D))

    degc = _sc_degrees(ssrc, sdst, e0, e1, zerosD)
    x_scaled = _tc_prescale(x_st, degc)
    agg1 = _sc_aggregate(x_scaled, gsrc, sdst, zerosD)
    pre2 = _tc_mid(agg1, degc, W1, b1.reshape(1, -1), W2)
    agg2 = _sc_aggregate(pre2, gsrc, sdst, zerosD)
    z = _tc_final(agg2, degc, b2.reshape(1, -1), Wp1, bp1.reshape(1, -1),
                  Wp2, bp2.reshape(1, -1))
    return (z[:N], z[N:])


# speedup vs baseline: 10.4033x; 1.3790x over previous
import functools

import jax
import jax.numpy as jnp
from jax import lax
from jax.experimental import pallas as pl
from jax.experimental.pallas import tpu as pltpu
from jax.experimental.pallas import tpu_sc as plsc

N = 10000
E = 320000
D = 128
NS = 16
K = 80

EPT_AGG = E // NS
NCH = EPT_AGG // K
GSC = 25
NSC = NCH // GSC
R_CHUNK = 624
R_TAIL = N - 15 * R_CHUNK
R_TC = 1000

_mesh = plsc.VectorSubcoreMesh(core_axis_name="c", subcore_axis_name="s")


def _rowsplit_copy(copy_fn, s):
    @pl.when(s < 15)
    def _():
        copy_fn(s * R_CHUNK, R_CHUNK)

    @pl.when(s == 15)
    def _():
        copy_fn(15 * R_CHUNK, R_TAIL)


@functools.partial(
    pl.kernel,
    mesh=_mesh,
    compiler_params=pltpu.CompilerParams(needs_layout_passes=False),
    out_type=jax.ShapeDtypeStruct((2, 2, NS, N), jnp.float32),
    scratch_types=[pltpu.VMEM((EPT_AGG,), jnp.int32),
                   pltpu.VMEM((EPT_AGG,), jnp.int32),
                   pltpu.VMEM((N,), jnp.float32),
                   pltpu.VMEM((N,), jnp.float32)],
)
def _sc_degrees(src_hbm, dst_hbm, hist_hbm, sidx_v, didx_v, ho_v, hi_v):
    c = lax.axis_index("c")
    s = lax.axis_index("s")
    zeros16 = jnp.zeros((16,), jnp.float32)
    ones16 = jnp.ones((16,), jnp.float32)

    def zbody(i, carry):
        ho_v[pl.ds(i * 16, 16)] = zeros16
        hi_v[pl.ds(i * 16, 16)] = zeros16
        return carry

    lax.fori_loop(0, N // 16, zbody, None)
    pltpu.sync_copy(src_hbm.at[c, s], sidx_v)
    pltpu.sync_copy(dst_hbm.at[c, s], didx_v)

    def body(j, carry):
        plsc.addupdate_scatter(ho_v, [sidx_v[pl.ds(j * 16, 16)]], ones16)
        plsc.addupdate_scatter(hi_v, [didx_v[pl.ds(j * 16, 16)]], ones16)
        return carry

    lax.fori_loop(0, EPT_AGG // 16, body, None)
    pltpu.sync_copy(ho_v, hist_hbm.at[c, 0, s])
    pltpu.sync_copy(hi_v, hist_hbm.at[c, 1, s])


@functools.partial(
    pl.kernel,
    mesh=_mesh,
    out_type=jax.ShapeDtypeStruct((2 * N, D), jnp.float32),
    scratch_types=[pltpu.VMEM((GSC, K), jnp.int32),
                   pltpu.VMEM((GSC, K), jnp.int32),
                   pltpu.VMEM((K, D), jnp.float32),
                   pltpu.VMEM((K, D), jnp.float32),
                   pltpu.VMEM((K, D), jnp.float32),
                   pltpu.VMEM((K, D), jnp.float32),
                   pltpu.VMEM_SHARED((N, D), jnp.float32),
                   pltpu.SemaphoreType.DMA,
                   pltpu.SemaphoreType.DMA,
                   pltpu.SemaphoreType.DMA,
                   pltpu.SemaphoreType.DMA,
                   pltpu.SemaphoreType.DMA,
                   pltpu.SemaphoreType.DMA],
)
def _sc_aggregate(x_hbm, gsrc_hbm, sdst_hbm, zeros_hbm, out_hbm,
                  sidx_v, didx_v, rows0_v, rows1_v, rows2_v, rows3_v, acc,
                  sem0, sem1, sem2, sem3, ssem0, ssem1):
    c = lax.axis_index("c")
    s = lax.axis_index("s")
    rows = (rows0_v, rows1_v, rows2_v, rows3_v)
    sems = (sem0, sem1, sem2, sem3)
    ssems = (ssem0, ssem1)

    def zero(r0, n):
        pltpu.sync_copy(zeros_hbm.at[pl.ds(r0, n)], acc.at[pl.ds(r0, n)])

    _rowsplit_copy(zero, s)
    plsc.subcore_barrier()

    def gather(j, p):
        pltpu.async_copy(x_hbm.at[sidx_v.at[j]], rows[p], sems[p])

    def gather_wait(j, p):
        pltpu.make_async_copy(x_hbm.at[sidx_v.at[j]], rows[p], sems[p]).wait()

    def outer(g, carry):
        pltpu.sync_copy(gsrc_hbm.at[c, s, g], sidx_v)
        pltpu.sync_copy(sdst_hbm.at[c, s, g], didx_v)
        for p in range(3):
            gather(p, p)

        def body(j, carry2):
            for q in range(4):
                @pl.when((j >= 1) & ((j - 1) % 4 == q))
                def _(q=q):
                    pltpu.make_async_copy(rows[q], acc.at[didx_v.at[j - 1]],
                                          ssems[q % 2]).wait()

            for q in range(4):
                @pl.when((j + 3 < GSC) & ((j + 3) % 4 == q))
                def _(q=q):
                    gather(j + 3, q)

            for q in range(4):
                @pl.when(j % 4 == q)
                def _(q=q):
                    gather_wait(j, q)
                    pltpu.async_copy(rows[q], acc.at[didx_v.at[j]],
                                     ssems[q % 2], add=True)
            return carry2

        lax.fori_loop(0, GSC, body, None)
        for q in range(4):
            @pl.when((GSC - 1) % 4 == q)
            def _(q=q):
                pltpu.make_async_copy(rows[q], acc.at[didx_v.at[GSC - 1]],
                                      ssems[q % 2]).wait()
        return carry

    lax.fori_loop(0, NSC, outer, None)
    plsc.subcore_barrier()

    def writeout(r0, n):
        pltpu.sync_copy(acc.at[pl.ds(r0, n)],
                        out_hbm.at[pl.ds(c * N + r0, n)])

    _rowsplit_copy(writeout, s)


_NB = N // R_TC


def _deg_col(h_ref):
    return jnp.sum(h_ref[0, 0, :, :], axis=1, keepdims=True)


def _hist_spec(kind):
    return pl.BlockSpec((1, 1, R_TC, NS),
                        lambda i, kind=kind: (i // _NB, kind, i % _NB, 0))


def _tc_prescale_body(x_ref, ho_ref, o_ref):
    dinv = lax.rsqrt(jnp.maximum(_deg_col(ho_ref), 1.0))
    o_ref[:, :] = x_ref[:, :] * dinv


def _tc_prescale(x, hist):
    return pl.pallas_call(
        _tc_prescale_body,
        grid=((2 * N) // R_TC,),
        in_specs=[pl.BlockSpec((R_TC, D), lambda i: (i, 0)),
                  _hist_spec(0)],
        out_specs=pl.BlockSpec((R_TC, D), lambda i: (i, 0)),
        out_shape=jax.ShapeDtypeStruct((2 * N, D), jnp.float32),
    )(x, hist)


def _tc_mid_body(a_ref, hi_ref, ho_ref, w1_ref, b1_ref, w2_ref, o_ref):
    din = lax.rsqrt(jnp.maximum(_deg_col(hi_ref), 1.0))
    a = a_ref[:, :] * din
    h = jnp.dot(a, w1_ref[:, :], preferred_element_type=jnp.float32)
    h = jnp.maximum(h + b1_ref[:, :], 0.0)
    p = jnp.dot(h, w2_ref[:, :], preferred_element_type=jnp.float32)
    dout = lax.rsqrt(jnp.maximum(_deg_col(ho_ref), 1.0))
    o_ref[:, :] = p * dout


def _tc_mid(agg1, hist, W1, b1, W2):
    return pl.pallas_call(
        _tc_mid_body,
        grid=((2 * N) // R_TC,),
        in_specs=[pl.BlockSpec((R_TC, D), lambda i: (i, 0)),
                  _hist_spec(1),
                  _hist_spec(0),
                  pl.BlockSpec((D, 2 * D), lambda i: (0, 0)),
                  pl.BlockSpec((1, 2 * D), lambda i: (0, 0)),
                  pl.BlockSpec((2 * D, D), lambda i: (0, 0))],
        out_specs=pl.BlockSpec((R_TC, D), lambda i: (i, 0)),
        out_shape=jax.ShapeDtypeStruct((2 * N, D), jnp.float32),
    )(agg1, hist, hist, W1, b1, W2)


def _tc_final_body(a_ref, hi_ref, b2_ref, wp1_ref, bp1_ref, wp2_ref,
                   bp2_ref, o_ref):
    din = lax.rsqrt(jnp.maximum(_deg_col(hi_ref), 1.0))
    h2 = jnp.maximum(a_ref[:, :] * din + b2_ref[:, :], 0.0)
    t = jnp.dot(h2, wp1_ref[:, :], preferred_element_type=jnp.float32)
    t = t + bp1_ref[:, :]
    t = jnp.where(t > 0.0, t, jnp.exp(t) - 1.0)
    z = jnp.dot(t, wp2_ref[:, :], preferred_element_type=jnp.float32)
    o_ref[:, :] = z + bp2_ref[:, :]


def _tc_final(agg2, hist, b2, Wp1, bp1, Wp2, bp2):
    return pl.pallas_call(
        _tc_final_body,
        grid=((2 * N) // R_TC,),
        in_specs=[pl.BlockSpec((R_TC, D), lambda i: (i, 0)),
                  _hist_spec(1),
                  pl.BlockSpec((1, D), lambda i: (0, 0)),
                  pl.BlockSpec((D, D), lambda i: (0, 0)),
                  pl.BlockSpec((1, D), lambda i: (0, 0)),
                  pl.BlockSpec((D, D), lambda i: (0, 0)),
                  pl.BlockSpec((1, D), lambda i: (0, 0))],
        out_specs=pl.BlockSpec((R_TC, D), lambda i: (i, 0)),
        out_shape=jax.ShapeDtypeStruct((2 * N, D), jnp.float32),
    )(agg2, hist, b2, Wp1, bp1, Wp2, bp2)


def kernel(feat1, feat2, W1, b1, W2, b2, Wp1, bp1, Wp2, bp2,
           edge_index1, edge_index2):
    src1, dst1 = edge_index1[0], edge_index1[1]
    src2, dst2 = edge_index2[0], edge_index2[1]
    idx5 = (2, NS, NSC, GSC, K)
    gsrc = jnp.concatenate([src1, src2 + N]).reshape(idx5)
    ssrc = jnp.concatenate([src1, src2]).reshape(2, NS, EPT_AGG)
    sdst_flat = jnp.concatenate([dst1, dst2])
    sdst5 = sdst_flat.reshape(idx5)
    sdst3 = sdst_flat.reshape(2, NS, EPT_AGG)
    x_st = jnp.concatenate([feat1, feat2], axis=0)
    zerosD = jnp.zeros((N, D), jnp.float32)

    hist = jnp.transpose(_sc_degrees(ssrc, sdst3), (0, 1, 3, 2))
    x_scaled = _tc_prescale(x_st, hist)
    agg1 = _sc_aggregate(x_scaled, gsrc, sdst5, zerosD)
    pre2 = _tc_mid(agg1, hist, W1, b1.reshape(1, -1), W2)
    agg2 = _sc_aggregate(pre2, gsrc, sdst5, zerosD)
    z = _tc_final(agg2, hist, b2.reshape(1, -1), Wp1, bp1.reshape(1, -1),
                  Wp2, bp2.reshape(1, -1))
    return (z[:N], z[N:])


# speedup vs baseline: 10.7723x; 1.0355x over previous
import functools

import jax
import jax.numpy as jnp
from jax import lax
from jax.experimental import pallas as pl
from jax.experimental.pallas import tpu as pltpu
from jax.experimental.pallas import tpu_sc as plsc

N = 10000
E = 320000
D = 128
NS = 16
K = 80

EPT_AGG = E // NS
NCH = EPT_AGG // K
GSC = 25
NSC = NCH // GSC
R_CHUNK = 624
R_TAIL = N - 15 * R_CHUNK
R_TC = 2000

_mesh = plsc.VectorSubcoreMesh(core_axis_name="c", subcore_axis_name="s")


def _rowsplit_copy(copy_fn, s):
    @pl.when(s < 15)
    def _():
        copy_fn(s * R_CHUNK, R_CHUNK)

    @pl.when(s == 15)
    def _():
        copy_fn(15 * R_CHUNK, R_TAIL)


@functools.partial(
    pl.kernel,
    mesh=_mesh,
    compiler_params=pltpu.CompilerParams(needs_layout_passes=False),
    out_type=jax.ShapeDtypeStruct((2, 2, NS, N), jnp.float32),
    scratch_types=[pltpu.VMEM((EPT_AGG,), jnp.int32),
                   pltpu.VMEM((EPT_AGG,), jnp.int32),
                   pltpu.VMEM((N,), jnp.float32),
                   pltpu.VMEM((N,), jnp.float32)],
)
def _sc_degrees(src_hbm, dst_hbm, hist_hbm, sidx_v, didx_v, ho_v, hi_v):
    c = lax.axis_index("c")
    s = lax.axis_index("s")
    zeros16 = jnp.zeros((16,), jnp.float32)
    ones16 = jnp.ones((16,), jnp.float32)

    def zbody(i, carry):
        ho_v[pl.ds(i * 16, 16)] = zeros16
        hi_v[pl.ds(i * 16, 16)] = zeros16
        return carry

    lax.fori_loop(0, N // 16, zbody, None)
    pltpu.sync_copy(src_hbm.at[c, s], sidx_v)
    pltpu.sync_copy(dst_hbm.at[c, s], didx_v)

    def body(j, carry):
        plsc.addupdate_scatter(ho_v, [sidx_v[pl.ds(j * 16, 16)]], ones16)
        plsc.addupdate_scatter(hi_v, [didx_v[pl.ds(j * 16, 16)]], ones16)
        return carry

    lax.fori_loop(0, EPT_AGG // 16, body, None)
    pltpu.sync_copy(ho_v, hist_hbm.at[c, 0, s])
    pltpu.sync_copy(hi_v, hist_hbm.at[c, 1, s])


@functools.partial(
    pl.kernel,
    mesh=_mesh,
    out_type=jax.ShapeDtypeStruct((2 * N, D), jnp.float32),
    scratch_types=[pltpu.VMEM((GSC, K), jnp.int32),
                   pltpu.VMEM((GSC, K), jnp.int32),
                   pltpu.VMEM((K, D), jnp.float32),
                   pltpu.VMEM((K, D), jnp.float32),
                   pltpu.VMEM((K, D), jnp.float32),
                   pltpu.VMEM((K, D), jnp.float32),
                   pltpu.VMEM_SHARED((N, D), jnp.float32),
                   pltpu.SemaphoreType.DMA,
                   pltpu.SemaphoreType.DMA,
                   pltpu.SemaphoreType.DMA,
                   pltpu.SemaphoreType.DMA,
                   pltpu.SemaphoreType.DMA,
                   pltpu.SemaphoreType.DMA],
)
def _sc_aggregate(x_hbm, gsrc_hbm, sdst_hbm, zeros_hbm, out_hbm,
                  sidx_v, didx_v, rows0_v, rows1_v, rows2_v, rows3_v, acc,
                  sem0, sem1, sem2, sem3, ssem0, ssem1):
    c = lax.axis_index("c")
    s = lax.axis_index("s")
    rows = (rows0_v, rows1_v, rows2_v, rows3_v)
    sems = (sem0, sem1, sem2, sem3)
    ssems = (ssem0, ssem1)

    def zero(r0, n):
        pltpu.sync_copy(zeros_hbm.at[pl.ds(r0, n)], acc.at[pl.ds(r0, n)])

    _rowsplit_copy(zero, s)
    plsc.subcore_barrier()

    def gather(j, p):
        pltpu.async_copy(x_hbm.at[sidx_v.at[j]], rows[p], sems[p])

    def gather_wait(j, p):
        pltpu.make_async_copy(x_hbm.at[sidx_v.at[j]], rows[p], sems[p]).wait()

    def outer(g, carry):
        pltpu.sync_copy(gsrc_hbm.at[c, s, g], sidx_v)
        pltpu.sync_copy(sdst_hbm.at[c, s, g], didx_v)
        for p in range(3):
            gather(p, p)

        def body(j, carry2):
            for q in range(4):
                @pl.when((j >= 1) & ((j - 1) % 4 == q))
                def _(q=q):
                    pltpu.make_async_copy(rows[q], acc.at[didx_v.at[j - 1]],
                                          ssems[q % 2]).wait()

            for q in range(4):
                @pl.when((j + 3 < GSC) & ((j + 3) % 4 == q))
                def _(q=q):
                    gather(j + 3, q)

            for q in range(4):
                @pl.when(j % 4 == q)
                def _(q=q):
                    gather_wait(j, q)
                    pltpu.async_copy(rows[q], acc.at[didx_v.at[j]],
                                     ssems[q % 2], add=True)
            return carry2

        lax.fori_loop(0, GSC, body, None)
        for q in range(4):
            @pl.when((GSC - 1) % 4 == q)
            def _(q=q):
                pltpu.make_async_copy(rows[q], acc.at[didx_v.at[GSC - 1]],
                                      ssems[q % 2]).wait()
        return carry

    lax.fori_loop(0, NSC, outer, None)
    plsc.subcore_barrier()

    def writeout(r0, n):
        pltpu.sync_copy(acc.at[pl.ds(r0, n)],
                        out_hbm.at[pl.ds(c * N + r0, n)])

    _rowsplit_copy(writeout, s)


_NB = N // R_TC


def _deg_col(h_ref):
    return jnp.sum(h_ref[0, 0, :, :], axis=1, keepdims=True)


def _hist_spec(kind):
    return pl.BlockSpec((1, 1, R_TC, NS),
                        lambda i, kind=kind: (i // _NB, kind, i % _NB, 0))


def _tc_prescale_body(x_ref, ho_ref, o_ref):
    dinv = lax.rsqrt(jnp.maximum(_deg_col(ho_ref), 1.0))
    o_ref[:, :] = x_ref[:, :] * dinv


def _tc_prescale(x, hist):
    return pl.pallas_call(
        _tc_prescale_body,
        grid=((2 * N) // R_TC,),
        in_specs=[pl.BlockSpec((R_TC, D), lambda i: (i, 0)),
                  _hist_spec(0)],
        out_specs=pl.BlockSpec((R_TC, D), lambda i: (i, 0)),
        out_shape=jax.ShapeDtypeStruct((2 * N, D), jnp.float32),
    )(x, hist)


def _tc_mid_body(a_ref, hi_ref, ho_ref, w1_ref, b1_ref, w2_ref, o_ref):
    din = lax.rsqrt(jnp.maximum(_deg_col(hi_ref), 1.0))
    a = a_ref[:, :] * din
    h = jnp.dot(a, w1_ref[:, :], preferred_element_type=jnp.float32)
    h = jnp.maximum(h + b1_ref[:, :], 0.0)
    p = jnp.dot(h, w2_ref[:, :], preferred_element_type=jnp.float32)
    dout = lax.rsqrt(jnp.maximum(_deg_col(ho_ref), 1.0))
    o_ref[:, :] = p * dout


def _tc_mid(agg1, hist, W1, b1, W2):
    return pl.pallas_call(
        _tc_mid_body,
        grid=((2 * N) // R_TC,),
        in_specs=[pl.BlockSpec((R_TC, D), lambda i: (i, 0)),
                  _hist_spec(1),
                  _hist_spec(0),
                  pl.BlockSpec((D, 2 * D), lambda i: (0, 0)),
                  pl.BlockSpec((1, 2 * D), lambda i: (0, 0)),
                  pl.BlockSpec((2 * D, D), lambda i: (0, 0))],
        out_specs=pl.BlockSpec((R_TC, D), lambda i: (i, 0)),
        out_shape=jax.ShapeDtypeStruct((2 * N, D), jnp.float32),
    )(agg1, hist, hist, W1, b1, W2)


def _tc_final_body(a_ref, hi_ref, b2_ref, wp1_ref, bp1_ref, wp2_ref,
                   bp2_ref, o_ref):
    din = lax.rsqrt(jnp.maximum(_deg_col(hi_ref), 1.0))
    h2 = jnp.maximum(a_ref[:, :] * din + b2_ref[:, :], 0.0)
    t = jnp.dot(h2, wp1_ref[:, :], preferred_element_type=jnp.float32)
    t = t + bp1_ref[:, :]
    t = jnp.where(t > 0.0, t, jnp.exp(t) - 1.0)
    z = jnp.dot(t, wp2_ref[:, :], preferred_element_type=jnp.float32)
    o_ref[:, :] = z + bp2_ref[:, :]


def _tc_final(agg2, hist, b2, Wp1, bp1, Wp2, bp2):
    return pl.pallas_call(
        _tc_final_body,
        grid=((2 * N) // R_TC,),
        in_specs=[pl.BlockSpec((R_TC, D), lambda i: (i, 0)),
                  _hist_spec(1),
                  pl.BlockSpec((1, D), lambda i: (0, 0)),
                  pl.BlockSpec((D, D), lambda i: (0, 0)),
                  pl.BlockSpec((1, D), lambda i: (0, 0)),
                  pl.BlockSpec((D, D), lambda i: (0, 0)),
                  pl.BlockSpec((1, D), lambda i: (0, 0))],
        out_specs=pl.BlockSpec((R_TC, D), lambda i: (i, 0)),
        out_shape=jax.ShapeDtypeStruct((2 * N, D), jnp.float32),
    )(agg2, hist, b2, Wp1, bp1, Wp2, bp2)


def kernel(feat1, feat2, W1, b1, W2, b2, Wp1, bp1, Wp2, bp2,
           edge_index1, edge_index2):
    src1, dst1 = edge_index1[0], edge_index1[1]
    src2, dst2 = edge_index2[0], edge_index2[1]
    idx5 = (2, NS, NSC, GSC, K)
    gsrc = jnp.concatenate([src1, src2 + N]).reshape(idx5)
    ssrc = jnp.concatenate([src1, src2]).reshape(2, NS, EPT_AGG)
    sdst_flat = jnp.concatenate([dst1, dst2])
    sdst5 = sdst_flat.reshape(idx5)
    sdst3 = sdst_flat.reshape(2, NS, EPT_AGG)
    x_st = jnp.concatenate([feat1, feat2], axis=0)
    zerosD = jnp.zeros((N, D), jnp.float32)

    hist = jnp.transpose(_sc_degrees(ssrc, sdst3), (0, 1, 3, 2))
    x_scaled = _tc_prescale(x_st, hist)
    agg1 = _sc_aggregate(x_scaled, gsrc, sdst5, zerosD)
    pre2 = _tc_mid(agg1, hist, W1, b1.reshape(1, -1), W2)
    agg2 = _sc_aggregate(pre2, gsrc, sdst5, zerosD)
    z = _tc_final(agg2, hist, b2.reshape(1, -1), Wp1, bp1.reshape(1, -1),
                  Wp2, bp2.reshape(1, -1))
    return (z[:N], z[N:])


# speedup vs baseline: 11.2393x; 1.0433x over previous
import functools

import jax
import jax.numpy as jnp
from jax import lax
from jax.experimental import pallas as pl
from jax.experimental.pallas import tpu as pltpu
from jax.experimental.pallas import tpu_sc as plsc

N = 10000
E = 320000
D = 128
NS = 16
K = 80

EPT_AGG = E // NS
NCH = EPT_AGG // K
GSC = 50
NSC = NCH // GSC
R_CHUNK = 624
R_TAIL = N - 15 * R_CHUNK
R_TC = 2000

_mesh = plsc.VectorSubcoreMesh(core_axis_name="c", subcore_axis_name="s")


def _rowsplit_copy(copy_fn, s):
    @pl.when(s < 15)
    def _():
        copy_fn(s * R_CHUNK, R_CHUNK)

    @pl.when(s == 15)
    def _():
        copy_fn(15 * R_CHUNK, R_TAIL)


@functools.partial(
    pl.kernel,
    mesh=_mesh,
    compiler_params=pltpu.CompilerParams(needs_layout_passes=False),
    out_type=jax.ShapeDtypeStruct((2, 2, NS, N), jnp.float32),
    scratch_types=[pltpu.VMEM((EPT_AGG,), jnp.int32),
                   pltpu.VMEM((EPT_AGG,), jnp.int32),
                   pltpu.VMEM((N,), jnp.float32),
                   pltpu.VMEM((N,), jnp.float32)],
)
def _sc_degrees(src_hbm, dst_hbm, hist_hbm, sidx_v, didx_v, ho_v, hi_v):
    c = lax.axis_index("c")
    s = lax.axis_index("s")
    zeros16 = jnp.zeros((16,), jnp.float32)
    ones16 = jnp.ones((16,), jnp.float32)

    def zbody(i, carry):
        ho_v[pl.ds(i * 16, 16)] = zeros16
        hi_v[pl.ds(i * 16, 16)] = zeros16
        return carry

    lax.fori_loop(0, N // 16, zbody, None)
    pltpu.sync_copy(src_hbm.at[c, s], sidx_v)
    pltpu.sync_copy(dst_hbm.at[c, s], didx_v)

    def body(j, carry):
        plsc.addupdate_scatter(ho_v, [sidx_v[pl.ds(j * 16, 16)]], ones16)
        plsc.addupdate_scatter(hi_v, [didx_v[pl.ds(j * 16, 16)]], ones16)
        return carry

    lax.fori_loop(0, EPT_AGG // 16, body, None)
    pltpu.sync_copy(ho_v, hist_hbm.at[c, 0, s])
    pltpu.sync_copy(hi_v, hist_hbm.at[c, 1, s])


@functools.partial(
    pl.kernel,
    mesh=_mesh,
    out_type=jax.ShapeDtypeStruct((2 * N, D), jnp.float32),
    scratch_types=[pltpu.VMEM((GSC, K), jnp.int32),
                   pltpu.VMEM((GSC, K), jnp.int32),
                   pltpu.VMEM((K, D), jnp.float32),
                   pltpu.VMEM((K, D), jnp.float32),
                   pltpu.VMEM((K, D), jnp.float32),
                   pltpu.VMEM_SHARED((N, D), jnp.float32),
                   pltpu.SemaphoreType.DMA,
                   pltpu.SemaphoreType.DMA,
                   pltpu.SemaphoreType.DMA,
                   pltpu.SemaphoreType.DMA,
                   pltpu.SemaphoreType.DMA],
)
def _sc_aggregate(x_hbm, gsrc_hbm, sdst_hbm, zeros_hbm, out_hbm,
                  sidx_v, didx_v, rows0_v, rows1_v, rows2_v, acc,
                  sem0, sem1, sem2, ssem0, ssem1):
    c = lax.axis_index("c")
    s = lax.axis_index("s")
    rows = (rows0_v, rows1_v, rows2_v)
    sems = (sem0, sem1, sem2)
    ssems = (ssem0, ssem1)

    def zero(r0, n):
        pltpu.sync_copy(zeros_hbm.at[pl.ds(r0, n)], acc.at[pl.ds(r0, n)])

    _rowsplit_copy(zero, s)
    plsc.subcore_barrier()

    def gather(j, p):
        pltpu.async_copy(x_hbm.at[sidx_v.at[j]], rows[p], sems[p])

    def gather_wait(j, p):
        pltpu.make_async_copy(x_hbm.at[sidx_v.at[j]], rows[p], sems[p]).wait()

    def outer(g, carry):
        pltpu.sync_copy(gsrc_hbm.at[c, s, g], sidx_v)
        pltpu.sync_copy(sdst_hbm.at[c, s, g], didx_v)
        for p in range(2):
            gather(p, p)

        def body(j, carry2):
            for q in range(3):
                @pl.when((j >= 1) & ((j - 1) % 3 == q))
                def _(q=q):
                    pltpu.make_async_copy(rows[q], acc.at[didx_v.at[j - 1]],
                                          ssems[q % 2]).wait()

            for q in range(3):
                @pl.when((j + 2 < GSC) & ((j + 2) % 3 == q))
                def _(q=q):
                    gather(j + 2, q)

            for q in range(3):
                @pl.when(j % 3 == q)
                def _(q=q):
                    gather_wait(j, q)
                    pltpu.async_copy(rows[q], acc.at[didx_v.at[j]],
                                     ssems[q % 2], add=True)
            return carry2

        lax.fori_loop(0, GSC, body, None)
        for q in range(3):
            @pl.when((GSC - 1) % 3 == q)
            def _(q=q):
                pltpu.make_async_copy(rows[q], acc.at[didx_v.at[GSC - 1]],
                                      ssems[q % 2]).wait()
        return carry

    lax.fori_loop(0, NSC, outer, None)
    plsc.subcore_barrier()

    def writeout(r0, n):
        pltpu.sync_copy(acc.at[pl.ds(r0, n)],
                        out_hbm.at[pl.ds(c * N + r0, n)])

    _rowsplit_copy(writeout, s)


_NB = N // R_TC


def _deg_col(h_ref):
    return jnp.sum(h_ref[0, 0, :, :], axis=1, keepdims=True)


def _hist_spec(kind):
    return pl.BlockSpec((1, 1, R_TC, NS),
                        lambda i, kind=kind: (i // _NB, kind, i % _NB, 0))


def _tc_prescale_body(x_ref, ho_ref, o_ref):
    dinv = lax.rsqrt(jnp.maximum(_deg_col(ho_ref), 1.0))
    o_ref[:, :] = x_ref[:, :] * dinv


def _tc_prescale(x, hist):
    return pl.pallas_call(
        _tc_prescale_body,
        grid=((2 * N) // R_TC,),
        in_specs=[pl.BlockSpec((R_TC, D), lambda i: (i, 0)),
                  _hist_spec(0)],
        out_specs=pl.BlockSpec((R_TC, D), lambda i: (i, 0)),
        out_shape=jax.ShapeDtypeStruct((2 * N, D), jnp.float32),
    )(x, hist)


def _tc_mid_body(a_ref, hi_ref, ho_ref, w1_ref, b1_ref, w2_ref, o_ref):
    din = lax.rsqrt(jnp.maximum(_deg_col(hi_ref), 1.0))
    a = a_ref[:, :] * din
    h = jnp.dot(a, w1_ref[:, :], preferred_element_type=jnp.float32)
    h = jnp.maximum(h + b1_ref[:, :], 0.0)
    p = jnp.dot(h, w2_ref[:, :], preferred_element_type=jnp.float32)
    dout = lax.rsqrt(jnp.maximum(_deg_col(ho_ref), 1.0))
    o_ref[:, :] = p * dout


def _tc_mid(agg1, hist, W1, b1, W2):
    return pl.pallas_call(
        _tc_mid_body,
        grid=((2 * N) // R_TC,),
        in_specs=[pl.BlockSpec((R_TC, D), lambda i: (i, 0)),
                  _hist_spec(1),
                  _hist_spec(0),
                  pl.BlockSpec((D, 2 * D), lambda i: (0, 0)),
                  pl.BlockSpec((1, 2 * D), lambda i: (0, 0)),
                  pl.BlockSpec((2 * D, D), lambda i: (0, 0))],
        out_specs=pl.BlockSpec((R_TC, D), lambda i: (i, 0)),
        out_shape=jax.ShapeDtypeStruct((2 * N, D), jnp.float32),
    )(agg1, hist, hist, W1, b1, W2)


def _tc_final_body(a_ref, hi_ref, b2_ref, wp1_ref, bp1_ref, wp2_ref,
                   bp2_ref, o_ref):
    din = lax.rsqrt(jnp.maximum(_deg_col(hi_ref), 1.0))
    h2 = jnp.maximum(a_ref[:, :] * din + b2_ref[:, :], 0.0)
    t = jnp.dot(h2, wp1_ref[:, :], preferred_element_type=jnp.float32)
    t = t + bp1_ref[:, :]
    t = jnp.where(t > 0.0, t, jnp.exp(t) - 1.0)
    z = jnp.dot(t, wp2_ref[:, :], preferred_element_type=jnp.float32)
    o_ref[:, :] = z + bp2_ref[:, :]


def _tc_final(agg2, hist, b2, Wp1, bp1, Wp2, bp2):
    return pl.pallas_call(
        _tc_final_body,
        grid=((2 * N) // R_TC,),
        in_specs=[pl.BlockSpec((R_TC, D), lambda i: (i, 0)),
                  _hist_spec(1),
                  pl.BlockSpec((1, D), lambda i: (0, 0)),
                  pl.BlockSpec((D, D), lambda i: (0, 0)),
                  pl.BlockSpec((1, D), lambda i: (0, 0)),
                  pl.BlockSpec((D, D), lambda i: (0, 0)),
                  pl.BlockSpec((1, D), lambda i: (0, 0))],
        out_specs=pl.BlockSpec((R_TC, D), lambda i: (i, 0)),
        out_shape=jax.ShapeDtypeStruct((2 * N, D), jnp.float32),
    )(agg2, hist, b2, Wp1, bp1, Wp2, bp2)


def kernel(feat1, feat2, W1, b1, W2, b2, Wp1, bp1, Wp2, bp2,
           edge_index1, edge_index2):
    src1, dst1 = edge_index1[0], edge_index1[1]
    src2, dst2 = edge_index2[0], edge_index2[1]
    idx5 = (2, NS, NSC, GSC, K)
    gsrc = jnp.concatenate([src1, src2 + N]).reshape(idx5)
    ssrc = jnp.concatenate([src1, src2]).reshape(2, NS, EPT_AGG)
    sdst_flat = jnp.concatenate([dst1, dst2])
    sdst5 = sdst_flat.reshape(idx5)
    sdst3 = sdst_flat.reshape(2, NS, EPT_AGG)
    x_st = jnp.concatenate([feat1, feat2], axis=0)
    zerosD = jnp.zeros((N, D), jnp.float32)

    hist = jnp.transpose(_sc_degrees(ssrc, sdst3), (0, 1, 3, 2))
    x_scaled = _tc_prescale(x_st, hist)
    agg1 = _sc_aggregate(x_scaled, gsrc, sdst5, zerosD)
    pre2 = _tc_mid(agg1, hist, W1, b1.reshape(1, -1), W2)
    agg2 = _sc_aggregate(pre2, gsrc, sdst5, zerosD)
    z = _tc_final(agg2, hist, b2.reshape(1, -1), Wp1, bp1.reshape(1, -1),
                  Wp2, bp2.reshape(1, -1))
    return (z[:N], z[N:])


# speedup vs baseline: 11.4243x; 1.0165x over previous
import functools

import jax
import jax.numpy as jnp
from jax import lax
from jax.experimental import pallas as pl
from jax.experimental.pallas import tpu as pltpu
from jax.experimental.pallas import tpu_sc as plsc

N = 10000
E = 320000
D = 128
NS = 16
K = 80

EPT_AGG = E // NS
NCH = EPT_AGG // K
GSC = 50
NSC = NCH // GSC
R_CHUNK = 624
R_TAIL = N - 15 * R_CHUNK
R_TC = 2000

_mesh = plsc.VectorSubcoreMesh(core_axis_name="c", subcore_axis_name="s")


def _rowsplit_copy(copy_fn, s):
    @pl.when(s < 15)
    def _():
        copy_fn(s * R_CHUNK, R_CHUNK)

    @pl.when(s == 15)
    def _():
        copy_fn(15 * R_CHUNK, R_TAIL)


@functools.partial(
    pl.kernel,
    mesh=_mesh,
    compiler_params=pltpu.CompilerParams(needs_layout_passes=False),
    out_type=jax.ShapeDtypeStruct((2, 2, NS, N), jnp.float32),
    scratch_types=[pltpu.VMEM((EPT_AGG,), jnp.int32),
                   pltpu.VMEM((EPT_AGG,), jnp.int32),
                   pltpu.VMEM((N,), jnp.float32),
                   pltpu.VMEM((N,), jnp.float32)],
)
def _sc_degrees(src_hbm, dst_hbm, hist_hbm, sidx_v, didx_v, ho_v, hi_v):
    c = lax.axis_index("c")
    s = lax.axis_index("s")
    zeros16 = jnp.zeros((16,), jnp.float32)
    ones16 = jnp.ones((16,), jnp.float32)

    def zbody(i, carry):
        ho_v[pl.ds(i * 16, 16)] = zeros16
        hi_v[pl.ds(i * 16, 16)] = zeros16
        return carry

    lax.fori_loop(0, N // 16, zbody, None)
    pltpu.sync_copy(src_hbm.at[c, s], sidx_v)
    pltpu.sync_copy(dst_hbm.at[c, s], didx_v)

    def body(j, carry):
        plsc.addupdate_scatter(ho_v, [sidx_v[pl.ds(j * 16, 16)]], ones16)
        plsc.addupdate_scatter(hi_v, [didx_v[pl.ds(j * 16, 16)]], ones16)
        return carry

    lax.fori_loop(0, EPT_AGG // 16, body, None)
    pltpu.sync_copy(ho_v, hist_hbm.at[c, 0, s])
    pltpu.sync_copy(hi_v, hist_hbm.at[c, 1, s])


@functools.partial(
    pl.kernel,
    mesh=_mesh,
    out_type=jax.ShapeDtypeStruct((2 * N, D), jnp.float32),
    scratch_types=[pltpu.VMEM((GSC, K), jnp.int32),
                   pltpu.VMEM((GSC, K), jnp.int32),
                   pltpu.VMEM((K, D), jnp.float32),
                   pltpu.VMEM((K, D), jnp.float32),
                   pltpu.VMEM((K, D), jnp.float32),
                   pltpu.VMEM_SHARED((N, D), jnp.float32),
                   pltpu.SemaphoreType.DMA,
                   pltpu.SemaphoreType.DMA,
                   pltpu.SemaphoreType.DMA,
                   pltpu.SemaphoreType.DMA,
                   pltpu.SemaphoreType.DMA],
)
def _sc_aggregate(x_hbm, gsrc_hbm, sdst_hbm, out_hbm,
                  sidx_v, didx_v, rows0_v, rows1_v, rows2_v, acc,
                  sem0, sem1, sem2, ssem0, ssem1):
    c = lax.axis_index("c")
    s = lax.axis_index("s")
    rows = (rows0_v, rows1_v, rows2_v)
    sems = (sem0, sem1, sem2)
    ssems = (ssem0, ssem1)

    zeros16 = jnp.zeros((16,), jnp.float32)

    def zrow(i, carry):
        for l in range(D // 16):
            rows0_v[i, pl.ds(l * 16, 16)] = zeros16
        return carry

    lax.fori_loop(0, K, zrow, None)

    def zcopy(r0, nfull):
        def zc(i, carry):
            pltpu.sync_copy(rows0_v, acc.at[pl.ds(r0 + i * K, K)])
            return carry
        lax.fori_loop(0, nfull, zc, None)

    @pl.when(s < 15)
    def _():
        zcopy(s * R_CHUNK, 7)
        pltpu.sync_copy(rows0_v.at[pl.ds(0, R_CHUNK - 7 * K)],
                        acc.at[pl.ds(s * R_CHUNK + 7 * K, R_CHUNK - 7 * K)])

    @pl.when(s == 15)
    def _():
        zcopy(15 * R_CHUNK, R_TAIL // K)

    plsc.subcore_barrier()

    def gather(j, p):
        pltpu.async_copy(x_hbm.at[sidx_v.at[j]], rows[p], sems[p])

    def gather_wait(j, p):
        pltpu.make_async_copy(x_hbm.at[sidx_v.at[j]], rows[p], sems[p]).wait()

    def outer(g, carry):
        pltpu.sync_copy(gsrc_hbm.at[c, s, g], sidx_v)
        pltpu.sync_copy(sdst_hbm.at[c, s, g], didx_v)
        for p in range(2):
            gather(p, p)

        def body(j, carry2):
            for q in range(3):
                @pl.when((j >= 1) & ((j - 1) % 3 == q))
                def _(q=q):
                    pltpu.make_async_copy(rows[q], acc.at[didx_v.at[j - 1]],
                                          ssems[q % 2]).wait()

            for q in range(3):
                @pl.when((j + 2 < GSC) & ((j + 2) % 3 == q))
                def _(q=q):
                    gather(j + 2, q)

            for q in range(3):
                @pl.when(j % 3 == q)
                def _(q=q):
                    gather_wait(j, q)
                    pltpu.async_copy(rows[q], acc.at[didx_v.at[j]],
                                     ssems[q % 2], add=True)
            return carry2

        lax.fori_loop(0, GSC, body, None)
        for q in range(3):
            @pl.when((GSC - 1) % 3 == q)
            def _(q=q):
                pltpu.make_async_copy(rows[q], acc.at[didx_v.at[GSC - 1]],
                                      ssems[q % 2]).wait()
        return carry

    lax.fori_loop(0, NSC, outer, None)
    plsc.subcore_barrier()

    def writeout(r0, n):
        pltpu.sync_copy(acc.at[pl.ds(r0, n)],
                        out_hbm.at[pl.ds(c * N + r0, n)])

    _rowsplit_copy(writeout, s)


_NB = N // R_TC


def _deg_col(h_ref):
    return jnp.sum(h_ref[0, 0, :, :], axis=1, keepdims=True)


def _hist_spec(kind):
    return pl.BlockSpec((1, 1, R_TC, NS),
                        lambda i, kind=kind: (i // _NB, kind, i % _NB, 0))


def _tc_prescale_body(x_ref, ho_ref, o_ref):
    dinv = lax.rsqrt(jnp.maximum(_deg_col(ho_ref), 1.0))
    o_ref[:, :] = x_ref[:, :] * dinv


def _tc_prescale(x, hist):
    return pl.pallas_call(
        _tc_prescale_body,
        grid=((2 * N) // R_TC,),
        in_specs=[pl.BlockSpec((R_TC, D), lambda i: (i, 0)),
                  _hist_spec(0)],
        out_specs=pl.BlockSpec((R_TC, D), lambda i: (i, 0)),
        out_shape=jax.ShapeDtypeStruct((2 * N, D), jnp.float32),
    )(x, hist)


def _tc_mid_body(a_ref, hi_ref, ho_ref, w1_ref, b1_ref, w2_ref, o_ref):
    din = lax.rsqrt(jnp.maximum(_deg_col(hi_ref), 1.0))
    a = a_ref[:, :] * din
    h = jnp.dot(a, w1_ref[:, :], preferred_element_type=jnp.float32)
    h = jnp.maximum(h + b1_ref[:, :], 0.0)
    p = jnp.dot(h, w2_ref[:, :], preferred_element_type=jnp.float32)
    dout = lax.rsqrt(jnp.maximum(_deg_col(ho_ref), 1.0))
    o_ref[:, :] = p * dout


def _tc_mid(agg1, hist, W1, b1, W2):
    return pl.pallas_call(
        _tc_mid_body,
        grid=((2 * N) // R_TC,),
        in_specs=[pl.BlockSpec((R_TC, D), lambda i: (i, 0)),
                  _hist_spec(1),
                  _hist_spec(0),
                  pl.BlockSpec((D, 2 * D), lambda i: (0, 0)),
                  pl.BlockSpec((1, 2 * D), lambda i: (0, 0)),
                  pl.BlockSpec((2 * D, D), lambda i: (0, 0))],
        out_specs=pl.BlockSpec((R_TC, D), lambda i: (i, 0)),
        out_shape=jax.ShapeDtypeStruct((2 * N, D), jnp.float32),
    )(agg1, hist, hist, W1, b1, W2)


def _tc_final_body(a_ref, hi_ref, b2_ref, wp1_ref, bp1_ref, wp2_ref,
                   bp2_ref, o_ref):
    din = lax.rsqrt(jnp.maximum(_deg_col(hi_ref), 1.0))
    h2 = jnp.maximum(a_ref[:, :] * din + b2_ref[:, :], 0.0)
    t = jnp.dot(h2, wp1_ref[:, :], preferred_element_type=jnp.float32)
    t = t + bp1_ref[:, :]
    t = jnp.where(t > 0.0, t, jnp.exp(t) - 1.0)
    z = jnp.dot(t, wp2_ref[:, :], preferred_element_type=jnp.float32)
    o_ref[:, :] = z + bp2_ref[:, :]


def _tc_final(agg2, hist, b2, Wp1, bp1, Wp2, bp2):
    return pl.pallas_call(
        _tc_final_body,
        grid=((2 * N) // R_TC,),
        in_specs=[pl.BlockSpec((R_TC, D), lambda i: (i, 0)),
                  _hist_spec(1),
                  pl.BlockSpec((1, D), lambda i: (0, 0)),
                  pl.BlockSpec((D, D), lambda i: (0, 0)),
                  pl.BlockSpec((1, D), lambda i: (0, 0)),
                  pl.BlockSpec((D, D), lambda i: (0, 0)),
                  pl.BlockSpec((1, D), lambda i: (0, 0))],
        out_specs=pl.BlockSpec((R_TC, D), lambda i: (i, 0)),
        out_shape=jax.ShapeDtypeStruct((2 * N, D), jnp.float32),
    )(agg2, hist, b2, Wp1, bp1, Wp2, bp2)


def kernel(feat1, feat2, W1, b1, W2, b2, Wp1, bp1, Wp2, bp2,
           edge_index1, edge_index2):
    src1, dst1 = edge_index1[0], edge_index1[1]
    src2, dst2 = edge_index2[0], edge_index2[1]
    idx5 = (2, NS, NSC, GSC, K)
    gsrc = jnp.concatenate([src1, src2 + N]).reshape(idx5)
    ssrc = jnp.concatenate([src1, src2]).reshape(2, NS, EPT_AGG)
    sdst_flat = jnp.concatenate([dst1, dst2])
    sdst5 = sdst_flat.reshape(idx5)
    sdst3 = sdst_flat.reshape(2, NS, EPT_AGG)
    x_st = jnp.concatenate([feat1, feat2], axis=0)

    hist = jnp.transpose(_sc_degrees(ssrc, sdst3), (0, 1, 3, 2))
    x_scaled = _tc_prescale(x_st, hist)
    agg1 = _sc_aggregate(x_scaled, gsrc, sdst5)
    pre2 = _tc_mid(agg1, hist, W1, b1.reshape(1, -1), W2)
    agg2 = _sc_aggregate(pre2, gsrc, sdst5)
    z = _tc_final(agg2, hist, b2.reshape(1, -1), Wp1, bp1.reshape(1, -1),
                  Wp2, bp2.reshape(1, -1))
    return (z[:N], z[N:])
